# baseline (speedup 1.0000x reference)
import functools

import jax
import jax.numpy as jnp
from jax import lax
from jax.experimental import pallas as pl
from jax.experimental.pallas import tpu as pltpu
from jax.experimental.pallas import tpu_sc as plsc

N_NODES = 100000
IN_SIZE = 128
OUT_SIZE = 128
S = 10
B = 4096
M = B * (S + 1)

NW = 32
A_PS = (3072, 1024)
CHUNK = 128


def _phase_geom(ph):
    a_p = A_PS[ph]
    n_p = a_p * S
    a_pw = a_p // NW
    n_pw = n_p // NW
    r_pw = a_pw + n_pw
    a_start = sum(A_PS[:ph])
    n_start = a_start * S
    chunks = [(0, a_pw, True, 0)]
    off, row = a_pw, 0
    while row < n_pw:
        sz = min(CHUNK, n_pw - row)
        chunks.append((off, sz, False, row))
        off += sz
        row += sz
    return a_p, n_p, a_pw, n_pw, r_pw, a_start, n_start, chunks


def _sc_body(ph, emb_hbm, nodes_hbm, neight_hbm,
             self_s_hbm, agg_s_hbm, self_n_hbm, agg_n_hbm,
             idx_v, nidx_v, selfbuf, aggbuf, sem_g0, sem_g1, sem_w):
    a_p, n_p, a_pw, n_pw, r_pw, a_start, n_start, chunks = _phase_geom(ph)
    wid = lax.axis_index("s") * 2 + lax.axis_index("c")
    sems_g = (sem_g0, sem_g1)

    a_col = a_start + wid * a_pw
    n_col = B + n_start + wid * n_pw
    stg = [
        pltpu.async_copy(nodes_hbm.at[pl.ds(a_col, a_pw)],
                         idx_v.at[pl.ds(0, a_pw)], sem_w),
        pltpu.async_copy(nodes_hbm.at[pl.ds(n_col, n_pw)],
                         idx_v.at[pl.ds(a_pw, n_pw)], sem_w),
    ]
    for j in range(S):
        stg.append(pltpu.async_copy(neight_hbm.at[pl.ds(j * M + a_col, a_pw)],
                                    nidx_v.at[pl.ds(j * r_pw, a_pw)], sem_w))
        stg.append(pltpu.async_copy(neight_hbm.at[pl.ds(j * M + n_col, n_pw)],
                                    nidx_v.at[pl.ds(j * r_pw + a_pw, n_pw)],
                                    sem_w))
    for cp in stg:
        cp.wait()

    zero16 = jnp.zeros((16,), jnp.float32)

    def zero_agg(buf, size):
        def zrow(i, carry):
            for k in range(IN_SIZE // 16):
                aggbuf[buf, i, pl.ds(k * 16, 16)] = zero16
            return carry
        lax.fori_loop(0, size, zrow, 0)

    def fire(ch, buf):
        off, size, _, _ = ch
        cps = [pltpu.async_copy(
            emb_hbm.at[idx_v.at[pl.ds(off, size)]],
            selfbuf.at[buf, pl.ds(0, size), :], sems_g[buf])]
        for j in range(S):
            cps.append(pltpu.async_copy(
                emb_hbm.at[nidx_v.at[pl.ds(j * r_pw + off, size)]],
                aggbuf.at[buf, pl.ds(0, size), :], sems_g[buf], add=True))
        return cps

    nch = len(chunks)
    zero_agg(0, chunks[0][1])
    gath = {0: fire(chunks[0], 0)}
    writes = {}
    for c in range(nch):
        buf = c % 2
        nxt = 1 - buf
        if c >= 1:
            for cp in writes[c - 1]:
                cp.wait()
        if c + 1 < nch:
            zero_agg(nxt, chunks[c + 1][1])
            gath[c + 1] = fire(chunks[c + 1], nxt)
        for cp in gath[c]:
            cp.wait()
        _, size, is_a, lrow = chunks[c]
        if is_a:
            o_self, o_agg, row = self_s_hbm, agg_s_hbm, wid * a_pw + lrow
        else:
            o_self, o_agg, row = self_n_hbm, agg_n_hbm, wid * n_pw + lrow
        writes[c] = [
            pltpu.async_copy(selfbuf.at[buf, pl.ds(0, size), :],
                             o_self.at[pl.ds(row, size), :], sem_w),
            pltpu.async_copy(aggbuf.at[buf, pl.ds(0, size), :],
                             o_agg.at[pl.ds(row, size), :], sem_w),
        ]
    for cp in writes[nch - 1]:
        cp.wait()


def _make_sc(ph):
    a_p, n_p, a_pw, n_pw, r_pw, _, _, _ = _phase_geom(ph)
    mesh = plsc.VectorSubcoreMesh(core_axis_name="c", subcore_axis_name="s")
    return pl.kernel(
        functools.partial(_sc_body, ph),
        out_type=[
            jax.ShapeDtypeStruct((a_p, IN_SIZE), jnp.float32),
            jax.ShapeDtypeStruct((a_p, IN_SIZE), jnp.float32),
            jax.ShapeDtypeStruct((n_p, IN_SIZE), jnp.float32),
            jax.ShapeDtypeStruct((n_p, IN_SIZE), jnp.float32),
        ],
        mesh=mesh,
        scratch_types=[
            pltpu.VMEM((r_pw,), jnp.int32),
            pltpu.VMEM((S * r_pw,), jnp.int32),
            pltpu.VMEM((2, CHUNK, IN_SIZE), jnp.float32),
            pltpu.VMEM((2, CHUNK, IN_SIZE), jnp.float32),
            pltpu.SemaphoreType.DMA,
            pltpu.SemaphoreType.DMA,
            pltpu.SemaphoreType.DMA,
        ],
    )


def _tc_body(ss, sa, ns, na, w1a, w1b, w2a, w2b, o):
    f32 = jnp.float32
    h1s = jnp.maximum(
        jnp.dot(ss[:], w1a[:], preferred_element_type=f32)
        + jnp.dot(sa[:], w1b[:], preferred_element_type=f32), 0.0)
    h1n = jnp.maximum(
        jnp.dot(ns[:], w1a[:], preferred_element_type=f32)
        + jnp.dot(na[:], w1b[:], preferred_element_type=f32), 0.0)
    agg1 = jnp.sum(h1n.reshape(h1s.shape[0], S, OUT_SIZE), axis=1)
    o[:] = jnp.maximum(
        jnp.dot(h1s, w2a[:], preferred_element_type=f32)
        + jnp.dot(agg1, w2b[:], preferred_element_type=f32), 0.0)


_TB = 256


def _tc_call(a_p, self_s, agg_s, self_n, agg_n, w1a, w1b, w2a, w2b):
    wspec = pl.BlockSpec((IN_SIZE, OUT_SIZE), lambda p: (0, 0))
    return pl.pallas_call(
        _tc_body,
        grid=(a_p // _TB,),
        in_specs=[
            pl.BlockSpec((_TB, IN_SIZE), lambda p: (p, 0)),
            pl.BlockSpec((_TB, IN_SIZE), lambda p: (p, 0)),
            pl.BlockSpec((_TB * S, IN_SIZE), lambda p: (p, 0)),
            pl.BlockSpec((_TB * S, IN_SIZE), lambda p: (p, 0)),
            wspec, wspec, wspec, wspec,
        ],
        out_specs=pl.BlockSpec((_TB, OUT_SIZE), lambda p: (p, 0)),
        out_shape=jax.ShapeDtypeStruct((a_p, OUT_SIZE), jnp.float32),
    )(self_s, agg_s, self_n, agg_n, w1a, w1b, w2a, w2b)


def kernel(emb_table, W1, W2, node_batch, nodes1, neigh1, neigh2):
    neigh_t = neigh1.T.reshape(-1)
    w1a = W1[:, :IN_SIZE].T
    w1b = W1[:, IN_SIZE:].T * (1.0 / S)
    w2a = W2[:, :OUT_SIZE].T
    w2b = W2[:, OUT_SIZE:].T * (1.0 / S)
    outs = []
    for ph in range(len(A_PS)):
        self_s, agg_s, self_n, agg_n = _make_sc(ph)(
            emb_table, nodes1, neigh_t)
        outs.append(_tc_call(A_PS[ph], self_s, agg_s, self_n, agg_n,
                             w1a, w1b, w2a, w2b))
    return jnp.concatenate(outs, axis=0)


# speedup vs baseline: 1.0241x; 1.0241x over previous
import functools

import jax
import jax.numpy as jnp
from jax import lax
from jax.experimental import pallas as pl
from jax.experimental.pallas import tpu as pltpu
from jax.experimental.pallas import tpu_sc as plsc

N_NODES = 100000
IN_SIZE = 128
OUT_SIZE = 128
S = 10
B = 4096
M = B * (S + 1)

NW = 32
A_PS = (3072, 1024)
CHUNK = 128


def _phase_geom(ph):
    a_p = A_PS[ph]
    n_p = a_p * S
    a_pw = a_p // NW
    n_pw = n_p // NW
    r_pw = a_pw + n_pw
    a_start = sum(A_PS[:ph])
    n_start = a_start * S
    chunks = [(0, a_pw, True, 0)]
    off, row = a_pw, 0
    while row < n_pw:
        sz = min(CHUNK, n_pw - row)
        chunks.append((off, sz, False, row))
        off += sz
        row += sz
    return a_p, n_p, a_pw, n_pw, r_pw, a_start, n_start, chunks


def _sc_body(ph, emb_hbm, nodes_hbm, neight_hbm,
             self_s_hbm, agg_s_hbm, self_n_hbm, agg_n_hbm,
             idx_v, nidx_v, selfbuf, aggbuf, sem_g0, sem_g1, sem_w):
    a_p, n_p, a_pw, n_pw, r_pw, a_start, n_start, chunks = _phase_geom(ph)
    wid = lax.axis_index("s") * 2 + lax.axis_index("c")
    sems_g = (sem_g0, sem_g1)

    a_col = a_start + wid * a_pw
    n_col = B + n_start + wid * n_pw
    stg = [
        pltpu.async_copy(nodes_hbm.at[pl.ds(a_col, a_pw)],
                         idx_v.at[pl.ds(0, a_pw)], sem_w),
        pltpu.async_copy(nodes_hbm.at[pl.ds(n_col, n_pw)],
                         idx_v.at[pl.ds(a_pw, n_pw)], sem_w),
    ]
    for j in range(S):
        stg.append(pltpu.async_copy(neight_hbm.at[pl.ds(j * M + a_col, a_pw)],
                                    nidx_v.at[pl.ds(j * r_pw, a_pw)], sem_w))
        stg.append(pltpu.async_copy(neight_hbm.at[pl.ds(j * M + n_col, n_pw)],
                                    nidx_v.at[pl.ds(j * r_pw + a_pw, n_pw)],
                                    sem_w))
    for cp in stg:
        cp.wait()

    zero16 = jnp.zeros((16,), jnp.float32)

    def zero_agg(buf, size):
        def zrow(i, carry):
            for k in range(IN_SIZE // 16):
                aggbuf[buf, i, pl.ds(k * 16, 16)] = zero16
            return carry
        lax.fori_loop(0, size, zrow, 0)

    def fire(ch, buf):
        off, size, _, _ = ch
        cps = [pltpu.async_copy(
            emb_hbm.at[idx_v.at[pl.ds(off, size)]],
            selfbuf.at[buf, pl.ds(0, size), :], sems_g[buf])]
        for j in range(S):
            cps.append(pltpu.async_copy(
                emb_hbm.at[nidx_v.at[pl.ds(j * r_pw + off, size)]],
                aggbuf.at[buf, pl.ds(0, size), :], sems_g[buf], add=True))
        return cps

    nch = len(chunks)
    zero_agg(0, chunks[0][1])
    gath = {0: fire(chunks[0], 0)}
    writes = {}
    for c in range(nch):
        buf = c % 2
        nxt = 1 - buf
        if c >= 1:
            for cp in writes[c - 1]:
                cp.wait()
        if c + 1 < nch:
            zero_agg(nxt, chunks[c + 1][1])
            gath[c + 1] = fire(chunks[c + 1], nxt)
        for cp in gath[c]:
            cp.wait()
        _, size, is_a, lrow = chunks[c]
        if is_a:
            o_self, o_agg, row = self_s_hbm, agg_s_hbm, wid * a_pw + lrow
        else:
            o_self, o_agg, row = self_n_hbm, agg_n_hbm, wid * n_pw + lrow
        writes[c] = [
            pltpu.async_copy(selfbuf.at[buf, pl.ds(0, size), :],
                             o_self.at[pl.ds(row, size), :], sem_w),
            pltpu.async_copy(aggbuf.at[buf, pl.ds(0, size), :],
                             o_agg.at[pl.ds(row, size), :], sem_w),
        ]
    for cp in writes[nch - 1]:
        cp.wait()


def _make_sc(ph):
    a_p, n_p, a_pw, n_pw, r_pw, _, _, _ = _phase_geom(ph)
    mesh = plsc.VectorSubcoreMesh(core_axis_name="c", subcore_axis_name="s")
    return pl.kernel(
        functools.partial(_sc_body, ph),
        out_type=[
            jax.ShapeDtypeStruct((a_p, IN_SIZE), jnp.float32),
            jax.ShapeDtypeStruct((a_p, IN_SIZE), jnp.float32),
            jax.ShapeDtypeStruct((n_p, IN_SIZE), jnp.float32),
            jax.ShapeDtypeStruct((n_p, IN_SIZE), jnp.float32),
        ],
        mesh=mesh,
        scratch_types=[
            pltpu.VMEM((r_pw,), jnp.int32),
            pltpu.VMEM((S * r_pw,), jnp.int32),
            pltpu.VMEM((2, CHUNK, IN_SIZE), jnp.float32),
            pltpu.VMEM((2, CHUNK, IN_SIZE), jnp.float32),
            pltpu.SemaphoreType.DMA,
            pltpu.SemaphoreType.DMA,
            pltpu.SemaphoreType.DMA,
        ],
    )


def _tc_body(ss, sa, ns, na, w1a, w1b, w2a, w2b, o):
    f32 = jnp.float32
    h1s = jnp.maximum(
        jnp.dot(ss[:], w1a[:], preferred_element_type=f32)
        + jnp.dot(sa[:], w1b[:], preferred_element_type=f32), 0.0)
    h1n = jnp.maximum(
        jnp.dot(ns[:], w1a[:], preferred_element_type=f32)
        + jnp.dot(na[:], w1b[:], preferred_element_type=f32), 0.0)
    agg1 = jnp.sum(h1n.reshape(h1s.shape[0], S, OUT_SIZE), axis=1)
    o[:] = jnp.maximum(
        jnp.dot(h1s, w2a[:], preferred_element_type=f32)
        + jnp.dot(agg1, w2b[:], preferred_element_type=f32), 0.0)


_TB = 256


def _tc_call(a_p, self_s, agg_s, self_n, agg_n, w1a, w1b, w2a, w2b):
    wspec = pl.BlockSpec((IN_SIZE, OUT_SIZE), lambda p: (0, 0))
    return pl.pallas_call(
        _tc_body,
        grid=(a_p // _TB,),
        in_specs=[
            pl.BlockSpec((_TB, IN_SIZE), lambda p: (p, 0)),
            pl.BlockSpec((_TB, IN_SIZE), lambda p: (p, 0)),
            pl.BlockSpec((_TB * S, IN_SIZE), lambda p: (p, 0)),
            pl.BlockSpec((_TB * S, IN_SIZE), lambda p: (p, 0)),
            wspec, wspec, wspec, wspec,
        ],
        out_specs=pl.BlockSpec((_TB, OUT_SIZE), lambda p: (p, 0)),
        out_shape=jax.ShapeDtypeStruct((a_p, OUT_SIZE), jnp.float32),
    )(self_s, agg_s, self_n, agg_n, w1a, w1b, w2a, w2b)


def kernel(emb_table, W1, W2, node_batch, nodes1, neigh1, neigh2):
    neigh_t = neigh1.T.reshape(-1)
    w1a = W1[:, :IN_SIZE].T
    w1b = W1[:, IN_SIZE:].T * (1.0 / S)
    w2a = W2[:, :OUT_SIZE].T
    w2b = W2[:, OUT_SIZE:].T * (1.0 / S)
    outs = []
    prev = None
    for ph in range(len(A_PS)):
        nodes_in, neigh_in = nodes1, neigh_t
        if prev is not None:
            nodes_in, neigh_in, _ = lax.optimization_barrier(
                (nodes1, neigh_t, prev))
        self_s, agg_s, self_n, agg_n = _make_sc(ph)(
            emb_table, nodes_in, neigh_in)
        prev = agg_n
        outs.append(_tc_call(A_PS[ph], self_s, agg_s, self_n, agg_n,
                             w1a, w1b, w2a, w2b))
    return jnp.concatenate(outs, axis=0)
